# baseline (speedup 1.0000x reference)
import jax
import jax.numpy as jnp
from jax.experimental import pallas as pl
from jax.experimental.pallas import tpu as pltpu

_NHEAD = 4
_C_IN = 256
_C_MID = 128
_C_STACK = _NHEAD * _C_MID
_C_OUT_PAD = 32
_N_BLK = 1024


def _fused_kernel(x_ref, w1_ref, b1_ref, w2_ref, b2_ref, wf_ref, add_ref,
                  out_ref):
    x = x_ref[...]
    h1 = jnp.dot(w1_ref[...], x, preferred_element_type=jnp.float32)
    h1 = jnp.maximum(h1 + b1_ref[...], 0.0)
    out = add_ref[...]
    for i in range(_NHEAD):
        lo = i * _C_MID
        hi = lo + _C_MID
        h2 = jnp.dot(w2_ref[i], h1[lo:hi, :],
                     preferred_element_type=jnp.float32)
        h2 = jnp.maximum(h2 + b2_ref[lo:hi, :], 0.0)
        out = out + jnp.dot(wf_ref[:, lo:hi], h2,
                            preferred_element_type=jnp.float32)
    out_ref[...] = out


def _fold_cbr(W, b, g, be):
    s = g / jnp.sqrt(1.0 + 1e-5)
    return s[:, None] * W, s * b + be


def kernel(vote_features, aggregated_vote_xyz, params):
    B, C, K = vote_features.shape
    N = B * K

    heads = [params['center'], params['size'], params['heading'],
             params['sem']]
    w1s, b1s, w2s, b2s = [], [], [], []
    for p in heads:
        w1, b1 = _fold_cbr(p['W1'], p['b1'], p['g1'], p['be1'])
        w2, b2 = _fold_cbr(p['W2'], p['b2'], p['g2'], p['be2'])
        w1s.append(w1); b1s.append(b1); w2s.append(w2); b2s.append(b2)
    W1 = jnp.concatenate(w1s, axis=0)
    b1 = jnp.concatenate(b1s, axis=0)[:, None]
    W2 = jnp.stack(w2s, axis=0)
    b2 = jnp.concatenate(b2s, axis=0)[:, None]

    g = params['gmm']
    ps = params['sem']
    Wf = jnp.zeros((_C_OUT_PAD, _C_STACK), jnp.float32)
    Wf = Wf.at[0:3, 0:128].set(g['Wc'])
    Wf = Wf.at[3:6, 128:256].set(g['Ws'])
    Wf = Wf.at[6:8, 256:384].set(g['Wh'])
    Wf = Wf.at[8:28, 384:512].set(ps['W3'])
    bf = jnp.zeros((_C_OUT_PAD,), jnp.float32)
    bf = bf.at[0:3].set(g['bc'])
    bf = bf.at[3:6].set(g['bs'])
    bf = bf.at[6:8].set(g['bh'])
    bf = bf.at[8:28].set(ps['b3'])

    xyz_t = jnp.transpose(aggregated_vote_xyz.reshape(N, 3))
    add = jnp.broadcast_to(bf[:, None], (_C_OUT_PAD, N))
    add = add.at[0:3, :].add(xyz_t)

    x = jnp.transpose(vote_features, (1, 0, 2)).reshape(C, N)

    grid = (N // _N_BLK,)
    out = pl.pallas_call(
        _fused_kernel,
        grid=grid,
        in_specs=[
            pl.BlockSpec((C, _N_BLK), lambda j: (0, j)),
            pl.BlockSpec((_C_STACK, _C_IN), lambda j: (0, 0)),
            pl.BlockSpec((_C_STACK, 1), lambda j: (0, 0)),
            pl.BlockSpec((_NHEAD, _C_MID, _C_MID), lambda j: (0, 0, 0)),
            pl.BlockSpec((_C_STACK, 1), lambda j: (0, 0)),
            pl.BlockSpec((_C_OUT_PAD, _C_STACK), lambda j: (0, 0)),
            pl.BlockSpec((_C_OUT_PAD, _N_BLK), lambda j: (0, j)),
        ],
        out_specs=pl.BlockSpec((_C_OUT_PAD, _N_BLK), lambda j: (0, j)),
        out_shape=jax.ShapeDtypeStruct((_C_OUT_PAD, N), jnp.float32),
        compiler_params=pltpu.CompilerParams(
            dimension_semantics=("parallel",)),
    )(x, W1, b1, W2, b2, Wf, add)

    return jnp.transpose(out.reshape(_C_OUT_PAD, B, K), (1, 2, 0))[:, :, :28]


# speedup vs baseline: 1.1299x; 1.1299x over previous
import jax
import jax.numpy as jnp
from jax.experimental import pallas as pl
from jax.experimental.pallas import tpu as pltpu

_NHEAD = 4
_C_IN = 256
_C_MID = 128
_C_STACK = _NHEAD * _C_MID
_C_OUT_PAD = 32


def _fused_kernel(x_ref, w1t_ref, b1_ref, w2t_ref, b2_ref, wft_ref, add_ref,
                  out_ref):
    x = x_ref[0]
    h1t = jax.lax.dot_general(
        x, w1t_ref[...], dimension_numbers=(((0,), (0,)), ((), ())),
        preferred_element_type=jnp.float32)
    h1t = jnp.maximum(h1t + b1_ref[...], 0.0)
    out = add_ref[0]
    for i in range(_NHEAD):
        lo = i * _C_MID
        hi = lo + _C_MID
        h2t = jnp.dot(h1t[:, lo:hi], w2t_ref[i],
                      preferred_element_type=jnp.float32)
        h2t = jnp.maximum(h2t + b2_ref[:, lo:hi], 0.0)
        out = out + jnp.dot(h2t, wft_ref[lo:hi, :],
                            preferred_element_type=jnp.float32)
    out_ref[0] = out


def _fold_cbr(W, b, g, be):
    s = g / jnp.sqrt(1.0 + 1e-5)
    return s[:, None] * W, s * b + be


def kernel(vote_features, aggregated_vote_xyz, params):
    B, C, K = vote_features.shape

    heads = [params['center'], params['size'], params['heading'],
             params['sem']]
    w1s, b1s, w2s, b2s = [], [], [], []
    for p in heads:
        w1, b1 = _fold_cbr(p['W1'], p['b1'], p['g1'], p['be1'])
        w2, b2 = _fold_cbr(p['W2'], p['b2'], p['g2'], p['be2'])
        w1s.append(w1); b1s.append(b1); w2s.append(w2); b2s.append(b2)
    W1t = jnp.concatenate(w1s, axis=0).T
    b1 = jnp.concatenate(b1s, axis=0)[None, :]
    W2t = jnp.stack([w.T for w in w2s], axis=0)
    b2 = jnp.concatenate(b2s, axis=0)[None, :]

    g = params['gmm']
    ps = params['sem']
    Wft = jnp.zeros((_C_STACK, _C_OUT_PAD), jnp.float32)
    Wft = Wft.at[0:128, 0:3].set(g['Wc'].T)
    Wft = Wft.at[128:256, 3:6].set(g['Ws'].T)
    Wft = Wft.at[256:384, 6:8].set(g['Wh'].T)
    Wft = Wft.at[384:512, 8:28].set(ps['W3'].T)
    bf = jnp.zeros((_C_OUT_PAD,), jnp.float32)
    bf = bf.at[0:3].set(g['bc'])
    bf = bf.at[3:6].set(g['bs'])
    bf = bf.at[6:8].set(g['bh'])
    bf = bf.at[8:28].set(ps['b3'])

    add = jnp.pad(aggregated_vote_xyz, ((0, 0), (0, 0), (0, _C_OUT_PAD - 3)))
    add = add + bf[None, None, :]

    out = pl.pallas_call(
        _fused_kernel,
        grid=(B,),
        in_specs=[
            pl.BlockSpec((1, C, K), lambda b: (b, 0, 0)),
            pl.BlockSpec((_C_IN, _C_STACK), lambda b: (0, 0)),
            pl.BlockSpec((1, _C_STACK), lambda b: (0, 0)),
            pl.BlockSpec((_NHEAD, _C_MID, _C_MID), lambda b: (0, 0, 0)),
            pl.BlockSpec((1, _C_STACK), lambda b: (0, 0)),
            pl.BlockSpec((_C_STACK, _C_OUT_PAD), lambda b: (0, 0)),
            pl.BlockSpec((1, K, _C_OUT_PAD), lambda b: (b, 0, 0)),
        ],
        out_specs=pl.BlockSpec((1, K, _C_OUT_PAD), lambda b: (b, 0, 0)),
        out_shape=jax.ShapeDtypeStruct((B, K, _C_OUT_PAD), jnp.float32),
        compiler_params=pltpu.CompilerParams(
            dimension_semantics=("parallel",)),
    )(vote_features, W1t, b1, W2t, b2, Wft, add)

    return out[:, :, :28]


# speedup vs baseline: 1.7525x; 1.5511x over previous
import jax
import jax.numpy as jnp
from jax.experimental import pallas as pl
from jax.experimental.pallas import tpu as pltpu

_NHEAD = 4

_XT_W = (((0,), (1,)), ((), ()))
_HT_W = (((1,), (1,)), ((), ()))
_BN_SCALE = 1.0 / (1.0 + 1e-5) ** 0.5


def _cbr_t(ht, g_ref, b_ref, be_ref):
    s = g_ref[...] * _BN_SCALE
    return jnp.maximum(ht * s + (s * b_ref[...] + be_ref[...]), 0.0)


def _fused_kernel(x_ref, xyz_ref,
                  cW1, cb1, cg1, cbe1, cW2, cb2, cg2, cbe2,
                  sW1, sb1, sg1, sbe1, sW2, sb2, sg2, sbe2,
                  hW1, hb1, hg1, hbe1, hW2, hb2, hg2, hbe2,
                  mW1, mb1, mg1, mbe1, mW2, mb2, mg2, mbe2,
                  Wc, bc, Ws, bs, Wh, bh, W3, b3,
                  out_ref):
    x = x_ref[0]

    def head(W1, b1, g1, be1, W2, b2, g2, be2):
        h1t = jax.lax.dot_general(x, W1[...], _XT_W,
                                  preferred_element_type=jnp.float32)
        h1t = _cbr_t(h1t, g1, b1, be1)
        h2t = jax.lax.dot_general(h1t, W2[...], _HT_W,
                                  preferred_element_type=jnp.float32)
        return _cbr_t(h2t, g2, b2, be2)

    fc = head(cW1, cb1, cg1, cbe1, cW2, cb2, cg2, cbe2)
    fs = head(sW1, sb1, sg1, sbe1, sW2, sb2, sg2, sbe2)
    fh = head(hW1, hb1, hg1, hbe1, hW2, hb2, hg2, hbe2)
    so = head(mW1, mb1, mg1, mbe1, mW2, mb2, mg2, mbe2)

    center = jax.lax.dot_general(fc, Wc[...], _HT_W,
                                 preferred_element_type=jnp.float32)
    center = center + bc[...] + xyz_ref[0]
    size = jax.lax.dot_general(fs, Ws[...], _HT_W,
                               preferred_element_type=jnp.float32) + bs[...]
    heading = jax.lax.dot_general(fh, Wh[...], _HT_W,
                                  preferred_element_type=jnp.float32) + bh[...]
    semobj = jax.lax.dot_general(so, W3[...], _HT_W,
                                 preferred_element_type=jnp.float32) + b3[...]
    out_ref[0] = jnp.concatenate([center, size, heading, semobj], axis=1)


def kernel(vote_features, aggregated_vote_xyz, params):
    B, C, K = vote_features.shape

    def head_args(p):
        return [p['W1'], p['b1'][None, :], p['g1'][None, :], p['be1'][None, :],
                p['W2'], p['b2'][None, :], p['g2'][None, :], p['be2'][None, :]]

    g = params['gmm']
    ps = params['sem']
    args = ([vote_features, aggregated_vote_xyz]
            + head_args(params['center'])
            + head_args(params['size'])
            + head_args(params['heading'])
            + head_args(params['sem'])
            + [g['Wc'], g['bc'][None, :], g['Ws'], g['bs'][None, :],
               g['Wh'], g['bh'][None, :], ps['W3'], ps['b3'][None, :]])

    def const_spec(a):
        shp = a.shape
        return pl.BlockSpec(shp, lambda b: (0,) * len(shp))

    in_specs = ([pl.BlockSpec((1, C, K), lambda b: (b, 0, 0)),
                 pl.BlockSpec((1, K, 3), lambda b: (b, 0, 0))]
                + [const_spec(a) for a in args[2:]])

    out = pl.pallas_call(
        _fused_kernel,
        grid=(B,),
        in_specs=in_specs,
        out_specs=pl.BlockSpec((1, K, 28), lambda b: (b, 0, 0)),
        out_shape=jax.ShapeDtypeStruct((B, K, 28), jnp.float32),
        compiler_params=pltpu.CompilerParams(
            dimension_semantics=("parallel",)),
    )(*args)
    return out


# speedup vs baseline: 2.1259x; 1.2131x over previous
import jax
import jax.numpy as jnp
from jax.experimental import pallas as pl
from jax.experimental.pallas import tpu as pltpu

_XT_W = (((0,), (1,)), ((), ()))
_HT_W = (((1,), (1,)), ((), ()))
_BN_SCALE = 1.0 / (1.0 + 1e-5) ** 0.5
_OUT_PAD = 32


def _bn_relu(ht, g_ref, b_ref, be_ref):
    s = g_ref[...] * _BN_SCALE
    return jnp.maximum(ht * s + (s * b_ref[...] + be_ref[...]), 0.0)


def _fused_kernel(x_ref, xyz_ref,
                  cW1, cb1, cg1, cbe1, cW2, cb2, cg2, cbe2,
                  sW1, sb1, sg1, sbe1, sW2, sb2, sg2, sbe2,
                  hW1, hb1, hg1, hbe1, hW2, hb2, hg2, hbe2,
                  mW1, mb1, mg1, mbe1, mW2, mb2, mg2, mbe2,
                  Wc, bc, Ws, bs, Wh, bh, W3, b3,
                  out_ref):
    x = x_ref[0].astype(jnp.bfloat16)

    W1 = jnp.concatenate(
        [cW1[...], sW1[...], hW1[...], mW1[...]], axis=0).astype(jnp.bfloat16)
    h1 = jax.lax.dot_general(x, W1, _XT_W,
                             preferred_element_type=jnp.float32)
    g1 = jnp.concatenate([cg1[...], sg1[...], hg1[...], mg1[...]], axis=1)
    b1 = jnp.concatenate([cb1[...], sb1[...], hb1[...], mb1[...]], axis=1)
    be1 = jnp.concatenate([cbe1[...], sbe1[...], hbe1[...], mbe1[...]], axis=1)
    s1 = g1 * _BN_SCALE
    h1 = jnp.maximum(h1 * s1 + (s1 * b1 + be1), 0.0).astype(jnp.bfloat16)

    def head2(i, W2, b2, g2, be2):
        lo = i * 128
        h2 = jax.lax.dot_general(h1[:, lo:lo + 128], W2[...].astype(jnp.bfloat16),
                                 _HT_W, preferred_element_type=jnp.float32)
        return _bn_relu(h2, g2, b2, be2).astype(jnp.bfloat16)

    fc = head2(0, cW2, cb2, cg2, cbe2)
    fs = head2(1, sW2, sb2, sg2, sbe2)
    fh = head2(2, hW2, hb2, hg2, hbe2)
    so = head2(3, mW2, mb2, mg2, mbe2)

    def proj(f, W, row_lo, n_rows):
        Wp = jnp.pad(W[...], ((row_lo, _OUT_PAD - row_lo - n_rows), (0, 0)))
        return jax.lax.dot_general(f, Wp.astype(jnp.bfloat16), _HT_W,
                                   preferred_element_type=jnp.float32)

    out = (proj(fc, Wc, 0, 3) + proj(fs, Ws, 3, 3)
           + proj(fh, Wh, 6, 2) + proj(so, W3, 8, 20))
    bias = jnp.concatenate([bc[...], bs[...], bh[...], b3[...]], axis=1)
    bias = jnp.pad(bias, ((0, 0), (0, _OUT_PAD - 28)))
    xyz = jnp.pad(xyz_ref[0], ((0, 0), (0, _OUT_PAD - 3)))
    out = out + bias + xyz
    out_ref[0] = out[:, :28]


def kernel(vote_features, aggregated_vote_xyz, params):
    B, C, K = vote_features.shape

    def head_args(p):
        return [p['W1'], p['b1'][None, :], p['g1'][None, :], p['be1'][None, :],
                p['W2'], p['b2'][None, :], p['g2'][None, :], p['be2'][None, :]]

    g = params['gmm']
    ps = params['sem']
    args = ([vote_features, aggregated_vote_xyz]
            + head_args(params['center'])
            + head_args(params['size'])
            + head_args(params['heading'])
            + head_args(params['sem'])
            + [g['Wc'], g['bc'][None, :], g['Ws'], g['bs'][None, :],
               g['Wh'], g['bh'][None, :], ps['W3'], ps['b3'][None, :]])

    def const_spec(a):
        shp = a.shape
        return pl.BlockSpec(shp, lambda b: (0,) * len(shp))

    in_specs = ([pl.BlockSpec((1, C, K), lambda b: (b, 0, 0)),
                 pl.BlockSpec((1, K, 3), lambda b: (b, 0, 0))]
                + [const_spec(a) for a in args[2:]])

    out = pl.pallas_call(
        _fused_kernel,
        grid=(B,),
        in_specs=in_specs,
        out_specs=pl.BlockSpec((1, K, 28), lambda b: (b, 0, 0)),
        out_shape=jax.ShapeDtypeStruct((B, K, 28), jnp.float32),
        compiler_params=pltpu.CompilerParams(
            dimension_semantics=("parallel",)),
    )(*args)
    return out


# speedup vs baseline: 2.4800x; 1.1666x over previous
import jax
import jax.numpy as jnp
from jax.experimental import pallas as pl
from jax.experimental.pallas import tpu as pltpu

_XT_W = (((0,), (1,)), ((), ()))
_HT_W = (((1,), (1,)), ((), ()))
_BN_SCALE = 1.0 / (1.0 + 1e-5) ** 0.5
_OUT_PAD = 32


def _fused_kernel(x_ref, xyz_ref,
                  cW1, cb1, cg1, cbe1, cW2, cb2, cg2, cbe2,
                  sW1, sb1, sg1, sbe1, sW2, sb2, sg2, sbe2,
                  hW1, hb1, hg1, hbe1, hW2, hb2, hg2, hbe2,
                  mW1, mb1, mg1, mbe1, mW2, mb2, mg2, mbe2,
                  Wc, bc, Ws, bs, Wh, bh, W3, b3,
                  out_ref):
    B = x_ref.shape[0]

    g1 = jnp.concatenate([cg1[...], sg1[...], hg1[...], mg1[...]], axis=1)
    b1 = jnp.concatenate([cb1[...], sb1[...], hb1[...], mb1[...]], axis=1)
    be1 = jnp.concatenate([cbe1[...], sbe1[...], hbe1[...], mbe1[...]], axis=1)
    s1 = g1 * _BN_SCALE
    W1 = jnp.concatenate(
        [cW1[...], sW1[...], hW1[...], mW1[...]], axis=0)
    W1 = (W1 * s1[0][:, None]).astype(jnp.bfloat16)
    bias1 = s1 * b1 + be1

    def prep2(W2, b2, g2, be2):
        s = g2[...] * _BN_SCALE
        return (W2[...] * s[0][:, None]).astype(jnp.bfloat16), \
            s * b2[...] + be2[...]

    W2s = [prep2(cW2, cb2, cg2, cbe2), prep2(sW2, sb2, sg2, sbe2),
           prep2(hW2, hb2, hg2, hbe2), prep2(mW2, mb2, mg2, mbe2)]

    def padw(W, row_lo, n_rows):
        return jnp.pad(W[...],
                       ((row_lo, _OUT_PAD - row_lo - n_rows), (0, 0))
                       ).astype(jnp.bfloat16)

    Wf = [padw(Wc, 0, 3), padw(Ws, 3, 3), padw(Wh, 6, 2), padw(W3, 8, 20)]
    bias_f = jnp.concatenate([bc[...], bs[...], bh[...], b3[...]], axis=1)
    bias_f = jnp.pad(bias_f, ((0, 0), (0, _OUT_PAD - 28)))

    for b in range(B):
        x = x_ref[b].astype(jnp.bfloat16)
        h1 = jax.lax.dot_general(x, W1, _XT_W,
                                 preferred_element_type=jnp.float32)
        h1 = jnp.maximum(h1 + bias1, 0.0).astype(jnp.bfloat16)

        out = jnp.pad(xyz_ref[b], ((0, 0), (0, _OUT_PAD - 3))) + bias_f
        for i in range(4):
            W2, bias2 = W2s[i]
            h2 = jax.lax.dot_general(h1[:, i * 128:(i + 1) * 128], W2, _HT_W,
                                     preferred_element_type=jnp.float32)
            h2 = jnp.maximum(h2 + bias2, 0.0).astype(jnp.bfloat16)
            out = out + jax.lax.dot_general(
                h2, Wf[i], _HT_W, preferred_element_type=jnp.float32)
        out_ref[b] = out[:, :28]


def kernel(vote_features, aggregated_vote_xyz, params):
    B, C, K = vote_features.shape

    def head_args(p):
        return [p['W1'], p['b1'][None, :], p['g1'][None, :], p['be1'][None, :],
                p['W2'], p['b2'][None, :], p['g2'][None, :], p['be2'][None, :]]

    g = params['gmm']
    ps = params['sem']
    args = ([vote_features, aggregated_vote_xyz]
            + head_args(params['center'])
            + head_args(params['size'])
            + head_args(params['heading'])
            + head_args(params['sem'])
            + [g['Wc'], g['bc'][None, :], g['Ws'], g['bs'][None, :],
               g['Wh'], g['bh'][None, :], ps['W3'], ps['b3'][None, :]])

    out = pl.pallas_call(
        _fused_kernel,
        out_shape=jax.ShapeDtypeStruct((B, K, 28), jnp.float32),
    )(*args)
    return out
